# Initial kernel scaffold; baseline (speedup 1.0000x reference)
import jax
import jax.numpy as jnp
from jax.experimental import pallas as pl


def kernel(qry_feats, key_feats, Wq, Wk, Wr, W1, b1, W2s, W2r):
    raise NotImplementedError("write your pallas kernel here")


# speedup vs baseline: 3.1869x; 3.1869x over previous
import jax
import jax.numpy as jnp
import numpy as np
from jax.experimental import pallas as pl

NQ = 100
NK = 16384
D = 256
K = 1024
ITERS = 2

NQP = 104
RC = 8
BJ = 1024
NBLK = NK // BJ
NRC = NQP // RC

_MININT = np.int32(-2147483648)


def _sortkey(x):
    b = jax.lax.bitcast_convert_type(x, jnp.int32)
    return jnp.where(b >= 0, b, jnp.bitwise_xor(jnp.bitwise_not(b), _MININT))


def _qproj_kernel(qry_ref, wq_ref, wr_ref, w1_ref, b1_ref, q_ref, r_ref, qw1_ref):
    qf = qry_ref[...]
    q = jnp.dot(qf, wq_ref[...], preferred_element_type=jnp.float32)
    q_ref[...] = q
    r_ref[...] = jnp.dot(qf, wr_ref[...], preferred_element_type=jnp.float32)
    qw1_ref[...] = (
        jnp.dot(q, w1_ref[...], preferred_element_type=jnp.float32) + b1_ref[...]
    )


def _keyproj_kernel(key_ref, wk_ref, w1_ref, q_ref, r_ref,
                    seg_ref, ref_ref, kw1t_ref):
    kb = jnp.dot(key_ref[...], wk_ref[...], preferred_element_type=jnp.float32)
    dn = (((1,), (1,)), ((), ()))
    seg_ref[...] = jax.lax.dot_general(q_ref[...], kb, dn,
                                       preferred_element_type=jnp.float32)
    ref_ref[...] = jax.lax.dot_general(r_ref[...], kb, dn,
                                       preferred_element_type=jnp.float32)
    kw1t_ref[...] = jax.lax.dot_general(
        w1_ref[...], kb, (((0,), (1,)), ((), ())),
        preferred_element_type=jnp.float32).astype(jnp.bfloat16)


def _thresh_kernel(ref_ref, thr_ref, m_ref):
    keys = _sortkey(ref_ref[...])

    def bit_step(i, tu):
        b = 31 - i
        cand = tu | (jnp.int32(1) << b)
        cand_s = cand ^ _MININT
        cnt = jnp.sum((keys >= cand_s).astype(jnp.int32), axis=1, keepdims=True)
        return jnp.where(cnt >= K, cand, tu)

    tu = jax.lax.fori_loop(0, 32, bit_step, jnp.zeros((NQP, 1), jnp.int32))
    ts = tu ^ _MININT
    cnt_gt = jnp.sum((keys > ts).astype(jnp.int32), axis=1, keepdims=True)
    need = K - cnt_gt
    eqm = keys == ts
    col = jax.lax.broadcasted_iota(jnp.int32, (NQP, NK), 1)

    def m_step(i, lo_hi):
        lo, hi = lo_hi
        mid = (lo + hi) // 2
        c = jnp.sum((eqm & (col < mid)).astype(jnp.int32), axis=1, keepdims=True)
        take = c >= need
        return jnp.where(take, lo, mid + 1), jnp.where(take, mid, hi)

    _, hi = jax.lax.fori_loop(
        0, 15, m_step,
        (jnp.zeros((NQP, 1), jnp.int32), jnp.full((NQP, 1), NK, jnp.int32)))
    thr_ref[...] = ts
    m_ref[...] = hi


def _update_kernel(kw1t_ref, ref_ref, seg_ref, qw1_ref, thr_ref, m_ref,
                   w2s_ref, w2r_ref, segout_ref, refout_ref):
    j = pl.program_id(0)
    refb = ref_ref[...]
    keys = _sortkey(refb)
    thr = thr_ref[...]
    mm = m_ref[...]
    col = j * BJ + jax.lax.broadcasted_iota(jnp.int32, (RC, BJ), 1)
    sel = (keys > thr) | ((keys == thr) & (col < mm))

    kw1t = kw1t_ref[...]
    w2s = w2s_ref[...].astype(jnp.bfloat16)
    w2r = w2r_ref[...].astype(jnp.bfloat16)
    qt = jnp.transpose(qw1_ref[...]).astype(jnp.bfloat16)

    ds_rows = []
    dr_rows = []
    for rr in range(RC):
        h = jnp.tanh(kw1t + qt[:, rr:rr + 1])
        ds_rows.append(jnp.sum(h * w2s, axis=0, keepdims=True))
        dr_rows.append(jnp.sum(h * w2r, axis=0, keepdims=True))
    dseg = jnp.concatenate(ds_rows, axis=0).astype(jnp.float32)
    dref = jnp.concatenate(dr_rows, axis=0).astype(jnp.float32)

    segout_ref[...] = seg_ref[...] + jnp.where(sel, dseg, 0.0)
    refout_ref[...] = refb + jnp.where(sel, dref - 1.0e4, 0.0)


def kernel(qry_feats, key_feats, Wq, Wk, Wr, W1, b1, W2s, W2r):
    f32 = jnp.float32
    qry_p = jnp.pad(qry_feats, ((0, NQP - NQ), (0, 0)))
    b1r = b1.reshape(1, D)

    q, r, qw1 = pl.pallas_call(
        _qproj_kernel,
        out_shape=[jax.ShapeDtypeStruct((NQP, D), f32)] * 3,
    )(qry_p, Wq, Wr, W1, b1r)

    seg, ref, kw1t = pl.pallas_call(
        _keyproj_kernel,
        grid=(NBLK,),
        in_specs=[
            pl.BlockSpec((BJ, D), lambda j: (j, 0)),
            pl.BlockSpec((D, D), lambda j: (0, 0)),
            pl.BlockSpec((D, D), lambda j: (0, 0)),
            pl.BlockSpec((NQP, D), lambda j: (0, 0)),
            pl.BlockSpec((NQP, D), lambda j: (0, 0)),
        ],
        out_specs=[
            pl.BlockSpec((NQP, BJ), lambda j: (0, j)),
            pl.BlockSpec((NQP, BJ), lambda j: (0, j)),
            pl.BlockSpec((D, BJ), lambda j: (0, j)),
        ],
        out_shape=[
            jax.ShapeDtypeStruct((NQP, NK), f32),
            jax.ShapeDtypeStruct((NQP, NK), f32),
            jax.ShapeDtypeStruct((D, NK), jnp.bfloat16),
        ],
    )(key_feats, Wk, W1, q, r)

    for _ in range(ITERS):
        thr, m = pl.pallas_call(
            _thresh_kernel,
            out_shape=[jax.ShapeDtypeStruct((NQP, 1), jnp.int32)] * 2,
        )(ref)

        seg, ref = pl.pallas_call(
            _update_kernel,
            grid=(NBLK, NRC),
            in_specs=[
                pl.BlockSpec((D, BJ), lambda j, c: (0, j)),
                pl.BlockSpec((RC, BJ), lambda j, c: (c, j)),
                pl.BlockSpec((RC, BJ), lambda j, c: (c, j)),
                pl.BlockSpec((RC, D), lambda j, c: (c, 0)),
                pl.BlockSpec((RC, 1), lambda j, c: (c, 0)),
                pl.BlockSpec((RC, 1), lambda j, c: (c, 0)),
                pl.BlockSpec((D, 1), lambda j, c: (0, 0)),
                pl.BlockSpec((D, 1), lambda j, c: (0, 0)),
            ],
            out_specs=[
                pl.BlockSpec((RC, BJ), lambda j, c: (c, j)),
                pl.BlockSpec((RC, BJ), lambda j, c: (c, j)),
            ],
            out_shape=[
                jax.ShapeDtypeStruct((NQP, NK), f32),
                jax.ShapeDtypeStruct((NQP, NK), f32),
            ],
        )(kw1t, ref, seg, qw1, thr, m, W2s, W2r)

    return seg[:NQ]


# speedup vs baseline: 4.6506x; 1.4593x over previous
import jax
import jax.numpy as jnp
import numpy as np
from jax.experimental import pallas as pl

NQ = 100
NK = 16384
D = 256
K = 1024
ITERS = 2

NQP = 104
RC = 8
BJ = 1024
NBLK = NK // BJ
NRC = NQP // RC

_MININT = np.int32(-2147483648)


def _sortkey(x):
    b = jax.lax.bitcast_convert_type(x, jnp.int32)
    return jnp.where(b >= 0, b, jnp.bitwise_xor(jnp.bitwise_not(b), _MININT))


def _qproj_kernel(qry_ref, wq_ref, wr_ref, w1_ref, b1_ref, q_ref, r_ref, qw1_ref):
    qf = qry_ref[...]
    q = jnp.dot(qf, wq_ref[...], preferred_element_type=jnp.float32)
    q_ref[...] = q
    r_ref[...] = jnp.dot(qf, wr_ref[...], preferred_element_type=jnp.float32)
    qw1_ref[...] = (
        jnp.dot(q, w1_ref[...], preferred_element_type=jnp.float32) + b1_ref[...]
    )


def _keyproj_kernel(key_ref, wk_ref, w1_ref, q_ref, r_ref,
                    seg_ref, ref_ref, kw1t_ref):
    kb = jnp.dot(key_ref[...], wk_ref[...], preferred_element_type=jnp.float32)
    dn = (((1,), (1,)), ((), ()))
    seg_ref[...] = jax.lax.dot_general(q_ref[...], kb, dn,
                                       preferred_element_type=jnp.float32)
    ref_ref[...] = jax.lax.dot_general(r_ref[...], kb, dn,
                                       preferred_element_type=jnp.float32)
    kw1t_ref[...] = jax.lax.dot_general(
        w1_ref[...], kb, (((0,), (1,)), ((), ())),
        preferred_element_type=jnp.float32).astype(jnp.bfloat16)


def _thresh_kernel(ref_ref, thr_ref, m_ref):
    keys = _sortkey(ref_ref[...])

    def bit_step(i, tu):
        b = 31 - i
        cand = tu | (jnp.int32(1) << b)
        cand_s = cand ^ _MININT
        cnt = jnp.sum((keys >= cand_s).astype(jnp.int32), axis=1, keepdims=True)
        return jnp.where(cnt >= K, cand, tu)

    tu = jax.lax.fori_loop(0, 32, bit_step, jnp.zeros((NQP, 1), jnp.int32))
    ts = tu ^ _MININT
    cnt_gt = jnp.sum((keys > ts).astype(jnp.int32), axis=1, keepdims=True)
    need = K - cnt_gt
    eqm = keys == ts
    col = jax.lax.broadcasted_iota(jnp.int32, (NQP, NK), 1)

    def m_step(i, lo_hi):
        lo, hi = lo_hi
        mid = (lo + hi) // 2
        c = jnp.sum((eqm & (col < mid)).astype(jnp.int32), axis=1, keepdims=True)
        take = c >= need
        return jnp.where(take, lo, mid + 1), jnp.where(take, mid, hi)

    _, hi = jax.lax.fori_loop(
        0, 15, m_step,
        (jnp.zeros((NQP, 1), jnp.int32), jnp.full((NQP, 1), NK, jnp.int32)))
    thr_ref[...] = ts
    m_ref[...] = hi


def _update_kernel(kw1t_ref, ref_ref, seg_ref, qw1_ref, thr_ref, m_ref,
                   w2p_ref, segout_ref, refout_ref):
    j = pl.program_id(0)
    refb = ref_ref[...]
    keys = _sortkey(refb)
    thr = thr_ref[...]
    mm = m_ref[...]
    col = j * BJ + jax.lax.broadcasted_iota(jnp.int32, (RC, BJ), 1)
    sel = (keys > thr) | ((keys == thr) & (col < mm))

    kw1t = kw1t_ref[...]
    w2p = w2p_ref[...].astype(jnp.bfloat16)
    qt = jnp.transpose(qw1_ref[...]).astype(jnp.bfloat16)

    ds_rows = []
    dr_rows = []
    for rr in range(RC):
        h = jnp.tanh(kw1t + qt[:, rr:rr + 1])
        y = jax.lax.dot_general(w2p, h, (((1,), (0,)), ((), ())),
                                preferred_element_type=jnp.float32)
        ds_rows.append(y[0:1])
        dr_rows.append(y[1:2])
    dseg = jnp.concatenate(ds_rows, axis=0)
    dref = jnp.concatenate(dr_rows, axis=0)

    segout_ref[...] = seg_ref[...] + jnp.where(sel, dseg, 0.0)
    refout_ref[...] = refb + jnp.where(sel, dref - 1.0e4, 0.0)


def kernel(qry_feats, key_feats, Wq, Wk, Wr, W1, b1, W2s, W2r):
    f32 = jnp.float32
    qry_p = jnp.pad(qry_feats, ((0, NQP - NQ), (0, 0)))
    b1r = b1.reshape(1, D)
    w2p = jnp.pad(jnp.concatenate([W2s, W2r], axis=1).T, ((0, 6), (0, 0)))

    q, r, qw1 = pl.pallas_call(
        _qproj_kernel,
        out_shape=[jax.ShapeDtypeStruct((NQP, D), f32)] * 3,
    )(qry_p, Wq, Wr, W1, b1r)

    seg, ref, kw1t = pl.pallas_call(
        _keyproj_kernel,
        grid=(NBLK,),
        in_specs=[
            pl.BlockSpec((BJ, D), lambda j: (j, 0)),
            pl.BlockSpec((D, D), lambda j: (0, 0)),
            pl.BlockSpec((D, D), lambda j: (0, 0)),
            pl.BlockSpec((NQP, D), lambda j: (0, 0)),
            pl.BlockSpec((NQP, D), lambda j: (0, 0)),
        ],
        out_specs=[
            pl.BlockSpec((NQP, BJ), lambda j: (0, j)),
            pl.BlockSpec((NQP, BJ), lambda j: (0, j)),
            pl.BlockSpec((D, BJ), lambda j: (0, j)),
        ],
        out_shape=[
            jax.ShapeDtypeStruct((NQP, NK), f32),
            jax.ShapeDtypeStruct((NQP, NK), f32),
            jax.ShapeDtypeStruct((D, NK), jnp.bfloat16),
        ],
    )(key_feats, Wk, W1, q, r)

    for _ in range(ITERS):
        thr, m = pl.pallas_call(
            _thresh_kernel,
            out_shape=[jax.ShapeDtypeStruct((NQP, 1), jnp.int32)] * 2,
        )(ref)

        seg, ref = pl.pallas_call(
            _update_kernel,
            grid=(NBLK, NRC),
            in_specs=[
                pl.BlockSpec((D, BJ), lambda j, c: (0, j)),
                pl.BlockSpec((RC, BJ), lambda j, c: (c, j)),
                pl.BlockSpec((RC, BJ), lambda j, c: (c, j)),
                pl.BlockSpec((RC, D), lambda j, c: (c, 0)),
                pl.BlockSpec((RC, 1), lambda j, c: (c, 0)),
                pl.BlockSpec((RC, 1), lambda j, c: (c, 0)),
                pl.BlockSpec((8, D), lambda j, c: (0, 0)),
            ],
            out_specs=[
                pl.BlockSpec((RC, BJ), lambda j, c: (c, j)),
                pl.BlockSpec((RC, BJ), lambda j, c: (c, j)),
            ],
            out_shape=[
                jax.ShapeDtypeStruct((NQP, NK), f32),
                jax.ShapeDtypeStruct((NQP, NK), f32),
            ],
        )(kw1t, ref, seg, qw1, thr, m, w2p)

    return seg[:NQ]


# speedup vs baseline: 4.8663x; 1.0464x over previous
import functools

import jax
import jax.numpy as jnp
import numpy as np
from jax.experimental import pallas as pl
from jax.experimental.pallas import tpu as pltpu
from jax.experimental.pallas import tpu_sc as plsc

NQ = 100
NK = 16384
D = 256
K = 1024
ITERS = 2

NQP = 104
BJ = 1024
NBLK = NK // BJ
NW = 32
GCH = 256

_MININT = np.int32(-2147483648)


def _sortkey(x):
    b = jax.lax.bitcast_convert_type(x, jnp.int32)
    return jnp.where(b >= 0, b, jnp.bitwise_xor(jnp.bitwise_not(b), _MININT))


def _qproj_kernel(qry_ref, wq_ref, wr_ref, w1_ref, b1_ref, q_ref, r_ref, qw1_ref):
    qf = qry_ref[...]
    q = jnp.dot(qf, wq_ref[...], preferred_element_type=jnp.float32)
    q_ref[...] = q
    r_ref[...] = jnp.dot(qf, wr_ref[...], preferred_element_type=jnp.float32)
    qw1_ref[...] = (
        jnp.dot(q, w1_ref[...], preferred_element_type=jnp.float32) + b1_ref[...]
    )


def _keyproj_kernel(key_ref, wk_ref, w1_ref, q_ref, r_ref,
                    seg_ref, ref_ref, kw1_ref):
    kb = jnp.dot(key_ref[...], wk_ref[...], preferred_element_type=jnp.float32)
    dn = (((1,), (1,)), ((), ()))
    seg_ref[...] = jax.lax.dot_general(q_ref[...], kb, dn,
                                       preferred_element_type=jnp.float32)
    ref_ref[...] = jax.lax.dot_general(r_ref[...], kb, dn,
                                       preferred_element_type=jnp.float32)
    kw1_ref[...] = jnp.dot(kb, w1_ref[...], preferred_element_type=jnp.float32)


def _thresh_kernel(ref_ref, thr_ref, m_ref):
    keys = _sortkey(ref_ref[...])

    def bit_step(i, tu):
        b = 31 - i
        cand = tu | (jnp.int32(1) << b)
        cand_s = cand ^ _MININT
        cnt = jnp.sum((keys >= cand_s).astype(jnp.int32), axis=1, keepdims=True)
        return jnp.where(cnt >= K, cand, tu)

    tu = jax.lax.fori_loop(0, 32, bit_step, jnp.zeros((NQP, 1), jnp.int32))
    ts = tu ^ _MININT
    cnt_gt = jnp.sum((keys > ts).astype(jnp.int32), axis=1, keepdims=True)
    need = K - cnt_gt
    eqm = keys == ts
    col = jax.lax.broadcasted_iota(jnp.int32, (NQP, NK), 1)

    def m_step(i, lo_hi):
        lo, hi = lo_hi
        mid = (lo + hi) // 2
        c = jnp.sum((eqm & (col < mid)).astype(jnp.int32), axis=1, keepdims=True)
        take = c >= need
        return jnp.where(take, lo, mid + 1), jnp.where(take, mid, hi)

    _, hi = jax.lax.fori_loop(
        0, 15, m_step,
        (jnp.zeros((NQP, 1), jnp.int32), jnp.full((NQP, 1), NK, jnp.int32)))
    thr_ref[...] = ts
    m_ref[...] = hi


def _mlp_kernel(gath_ref, qw1_ref, w2p_ref, ds_ref, dr_ref):
    i = pl.program_id(0)
    qrow = qw1_ref[pl.ds(i, 1), :]
    g = gath_ref[...] + qrow
    h = jnp.tanh(g).astype(jnp.bfloat16)
    y = jax.lax.dot_general(h, w2p_ref[...].astype(jnp.bfloat16),
                            (((1,), (1,)), ((), ())),
                            preferred_element_type=jnp.float32)
    ds_ref[...] = jnp.transpose(y[:, 0:1]).reshape(1, 1, K)
    dr_ref[...] = jnp.transpose(y[:, 1:2]).reshape(1, 1, K)


def _sc_gather_kernel(ref_hbm, thr_hbm, m_hbm, kw1_hbm, idx_hbm, gath_hbm,
                      refrow_v, idx_v, gbuf_v, thr_s, m_s, sem):
    cid = jax.lax.axis_index("c")
    sid = jax.lax.axis_index("s")
    wid = sid * 2 + cid
    pltpu.sync_copy(thr_hbm, thr_s)
    pltpu.sync_copy(m_hbm, m_s)

    def do_row(row):
        pltpu.sync_copy(ref_hbm.at[row], refrow_v)
        rowvec = jnp.full((16,), row, jnp.int32)
        thr = plsc.load_gather(thr_s, [rowvec])
        mm = plsc.load_gather(m_s, [rowvec])

        def chunk(c, off):
            v = refrow_v[pl.ds(c * 16, 16)]
            b = jax.lax.bitcast_convert_type(v, jnp.int32)
            key = jnp.where(b >= 0, b,
                            jnp.bitwise_xor(jnp.bitwise_not(b), _MININT))
            cols = jax.lax.iota(jnp.int32, 16) + c * 16
            sel = (key > thr) | ((key == thr) & (cols < mm))
            plsc.store_compressed(idx_v.at[pl.ds(off, 16)], cols, mask=sel)
            return off + jnp.sum(sel.astype(jnp.int32))

        jax.lax.fori_loop(0, NK // 16, chunk, jnp.int32(0))
        pltpu.sync_copy(idx_v, idx_hbm.at[row])
        for h in range(K // GCH):
            pltpu.async_copy(
                kw1_hbm.at[idx_v.at[pl.ds(h * GCH, GCH)]], gbuf_v, sem).wait()
            pltpu.sync_copy(
                gbuf_v, gath_hbm.at[pl.ds(row * K + h * GCH, GCH)])

    for rb in range(4):
        row = wid + rb * NW

        @pl.when(row < NQP)
        def _():
            do_row(row)


def _sc_scatter_kernel(seg_hbm, ref_hbm, idx_hbm, ds_hbm, dr_hbm,
                       segout_hbm, refout_hbm,
                       segrow_v, refrow_v, idxrow_v, dsrow_v, drrow_v, sem):
    cid = jax.lax.axis_index("c")
    sid = jax.lax.axis_index("s")
    wid = sid * 2 + cid

    def do_row(row):
        pltpu.sync_copy(seg_hbm.at[row], segrow_v)
        pltpu.sync_copy(ref_hbm.at[row], refrow_v)
        pltpu.sync_copy(idx_hbm.at[row], idxrow_v)
        pltpu.sync_copy(ds_hbm.at[row], dsrow_v)
        pltpu.sync_copy(dr_hbm.at[row], drrow_v)

        def chunk(c, carry):
            iv = idxrow_v[pl.ds(c * 16, 16)]
            dsv = dsrow_v[pl.ds(c * 16, 16)]
            drv = drrow_v[pl.ds(c * 16, 16)] - 1.0e4
            plsc.addupdate_scatter(segrow_v, [iv], dsv)
            plsc.addupdate_scatter(refrow_v, [iv], drv)
            return carry

        jax.lax.fori_loop(0, K // 16, chunk, jnp.int32(0))
        pltpu.sync_copy(segrow_v, segout_hbm.at[row])
        pltpu.sync_copy(refrow_v, refout_hbm.at[row])

    for rb in range(4):
        row = wid + rb * NW

        @pl.when(row < NQP)
        def _():
            do_row(row)


def kernel(qry_feats, key_feats, Wq, Wk, Wr, W1, b1, W2s, W2r):
    f32 = jnp.float32
    i32 = jnp.int32
    qry_p = jnp.pad(qry_feats, ((0, NQP - NQ), (0, 0)))
    b1r = b1.reshape(1, D)
    w2p = jnp.pad(jnp.concatenate([W2s, W2r], axis=1).T, ((0, 6), (0, 0)))

    q, r, qw1 = pl.pallas_call(
        _qproj_kernel,
        out_shape=[jax.ShapeDtypeStruct((NQP, D), f32)] * 3,
    )(qry_p, Wq, Wr, W1, b1r)

    seg, ref, kw1 = pl.pallas_call(
        _keyproj_kernel,
        grid=(NBLK,),
        in_specs=[
            pl.BlockSpec((BJ, D), lambda j: (j, 0)),
            pl.BlockSpec((D, D), lambda j: (0, 0)),
            pl.BlockSpec((D, D), lambda j: (0, 0)),
            pl.BlockSpec((NQP, D), lambda j: (0, 0)),
            pl.BlockSpec((NQP, D), lambda j: (0, 0)),
        ],
        out_specs=[
            pl.BlockSpec((NQP, BJ), lambda j: (0, j)),
            pl.BlockSpec((NQP, BJ), lambda j: (0, j)),
            pl.BlockSpec((BJ, D), lambda j: (j, 0)),
        ],
        out_shape=[
            jax.ShapeDtypeStruct((NQP, NK), f32),
            jax.ShapeDtypeStruct((NQP, NK), f32),
            jax.ShapeDtypeStruct((NK, D), f32),
        ],
    )(key_feats, Wk, W1, q, r)

    vmesh = plsc.VectorSubcoreMesh(core_axis_name="c", subcore_axis_name="s")
    sc_params = pltpu.CompilerParams(needs_layout_passes=False)

    sc_gather = pl.kernel(
        _sc_gather_kernel,
        compiler_params=sc_params,
        out_type=[
            jax.ShapeDtypeStruct((NQP, K), i32),
            jax.ShapeDtypeStruct((NQP * K, D), f32),
        ],
        mesh=vmesh,
        scratch_types=[
            pltpu.VMEM((NK,), f32),
            pltpu.VMEM((K,), i32),
            pltpu.VMEM((GCH, D), f32),
            pltpu.VMEM((NQP,), i32),
            pltpu.VMEM((NQP,), i32),
            pltpu.SemaphoreType.DMA,
        ],
    )

    sc_scatter = pl.kernel(
        _sc_scatter_kernel,
        compiler_params=sc_params,
        out_type=[
            jax.ShapeDtypeStruct((NQP, NK), f32),
            jax.ShapeDtypeStruct((NQP, NK), f32),
        ],
        mesh=vmesh,
        scratch_types=[
            pltpu.VMEM((NK,), f32),
            pltpu.VMEM((NK,), f32),
            pltpu.VMEM((K,), i32),
            pltpu.VMEM((K,), f32),
            pltpu.VMEM((K,), f32),
            pltpu.SemaphoreType.DMA,
        ],
    )

    for _ in range(ITERS):
        thr, m = pl.pallas_call(
            _thresh_kernel,
            out_shape=[jax.ShapeDtypeStruct((NQP, 1), i32)] * 2,
        )(ref)

        idx, gath = sc_gather(ref, thr.reshape(NQP), m.reshape(NQP), kw1)

        ds3, dr3 = pl.pallas_call(
            _mlp_kernel,
            grid=(NQP,),
            in_specs=[
                pl.BlockSpec((K, D), lambda i: (i, 0)),
                pl.BlockSpec((NQP, D), lambda i: (0, 0)),
                pl.BlockSpec((8, D), lambda i: (0, 0)),
            ],
            out_specs=[
                pl.BlockSpec((1, 1, K), lambda i: (i, 0, 0)),
                pl.BlockSpec((1, 1, K), lambda i: (i, 0, 0)),
            ],
            out_shape=[
                jax.ShapeDtypeStruct((NQP, 1, K), f32),
                jax.ShapeDtypeStruct((NQP, 1, K), f32),
            ],
        )(gath, qw1, w2p)

        seg, ref = sc_scatter(seg, ref, idx,
                              ds3.reshape(NQP, K), dr3.reshape(NQP, K))

    return seg[:NQ]


# speedup vs baseline: 5.8178x; 1.1955x over previous
import functools

import jax
import jax.numpy as jnp
import numpy as np
from jax.experimental import pallas as pl
from jax.experimental.pallas import tpu as pltpu
from jax.experimental.pallas import tpu_sc as plsc

NQ = 100
NK = 16384
D = 256
K = 1024
ITERS = 2

NQP = 104
BJ = 1024
NBLK = NK // BJ
NW = 32
GCH = 256

_MININT = np.int32(-2147483648)


def _sortkey(x):
    b = jax.lax.bitcast_convert_type(x, jnp.int32)
    return jnp.where(b >= 0, b, jnp.bitwise_xor(jnp.bitwise_not(b), _MININT))


def _qproj_kernel(qry_ref, wq_ref, wr_ref, w1_ref, b1_ref, q_ref, r_ref, qw1_ref):
    qf = qry_ref[...]
    q = jnp.dot(qf, wq_ref[...], preferred_element_type=jnp.float32)
    q_ref[...] = q
    r_ref[...] = jnp.dot(qf, wr_ref[...], preferred_element_type=jnp.float32)
    qw1_ref[...] = (
        jnp.dot(q, w1_ref[...], preferred_element_type=jnp.float32) + b1_ref[...]
    )


def _keyproj_kernel(key_ref, wk_ref, w1_ref, q_ref, r_ref,
                    seg_ref, ref_ref, kw1p_ref):
    kb = jnp.dot(key_ref[...], wk_ref[...], preferred_element_type=jnp.float32)
    dn = (((1,), (1,)), ((), ()))
    seg_ref[...] = jax.lax.dot_general(q_ref[...], kb, dn,
                                       preferred_element_type=jnp.float32)
    ref_ref[...] = jax.lax.dot_general(r_ref[...], kb, dn,
                                       preferred_element_type=jnp.float32)
    kw1 = jnp.dot(kb, w1_ref[...], preferred_element_type=jnp.float32)
    ilo = jax.lax.bitcast_convert_type(kw1[:, :128], jnp.int32)
    ihi = jax.lax.bitcast_convert_type(kw1[:, 128:], jnp.int32)
    half = jnp.int32(0x8000)
    kw1p_ref[...] = (((ilo + half) >> 16) & jnp.int32(0xFFFF)) | (
        (ihi + half) & jnp.int32(-65536))


def _thresh_kernel(ref_ref, thr_ref, m_ref):
    keys = _sortkey(ref_ref[...])

    def bit_step(i, tu):
        b = 31 - i
        cand = tu | (jnp.int32(1) << b)
        cand_s = cand ^ _MININT
        cnt = jnp.sum((keys >= cand_s).astype(jnp.int32), axis=1, keepdims=True)
        return jnp.where(cnt >= K, cand, tu)

    tu = jax.lax.fori_loop(0, 32, bit_step, jnp.zeros((NQP, 1), jnp.int32))
    ts = tu ^ _MININT
    cnt_gt = jnp.sum((keys > ts).astype(jnp.int32), axis=1, keepdims=True)
    need = K - cnt_gt
    eqm = keys == ts
    col = jax.lax.broadcasted_iota(jnp.int32, (NQP, NK), 1)

    def m_step(i, lo_hi):
        lo, hi = lo_hi
        mid = (lo + hi) // 2
        c = jnp.sum((eqm & (col < mid)).astype(jnp.int32), axis=1, keepdims=True)
        take = c >= need
        return jnp.where(take, lo, mid + 1), jnp.where(take, mid, hi)

    _, hi = jax.lax.fori_loop(
        0, 15, m_step,
        (jnp.zeros((NQP, 1), jnp.int32), jnp.full((NQP, 1), NK, jnp.int32)))
    thr_ref[...] = ts
    m_ref[...] = hi


def _mlp_kernel(gath_ref, qw1_ref, w2lo_ref, w2hi_ref, ds_ref, dr_ref):
    i = pl.program_id(0)
    qrow = qw1_ref[pl.ds(i, 1), :]
    packed = gath_ref[...]
    x_lo = jax.lax.bitcast_convert_type(packed << 16, jnp.float32)
    x_hi = jax.lax.bitcast_convert_type(packed & jnp.int32(-65536),
                                        jnp.float32)
    h_lo = jnp.tanh(x_lo + qrow[:, :128]).astype(jnp.bfloat16)
    h_hi = jnp.tanh(x_hi + qrow[:, 128:]).astype(jnp.bfloat16)
    dn = (((1,), (1,)), ((), ()))
    y = (jax.lax.dot_general(h_lo, w2lo_ref[...].astype(jnp.bfloat16), dn,
                             preferred_element_type=jnp.float32)
         + jax.lax.dot_general(h_hi, w2hi_ref[...].astype(jnp.bfloat16), dn,
                               preferred_element_type=jnp.float32))
    ds_ref[...] = jnp.transpose(y[:, 0:1]).reshape(1, 1, K)
    dr_ref[...] = jnp.transpose(y[:, 1:2]).reshape(1, 1, K)


def _sc_gather_kernel(ref_hbm, thr_hbm, m_hbm, kw1_hbm, idx_hbm, gath_hbm,
                      refrow_v, idx_v, gbuf_a, gbuf_b, thr_s, m_s,
                      sga, sgb, swa, swb):
    cid = jax.lax.axis_index("c")
    sid = jax.lax.axis_index("s")
    wid = sid * 2 + cid
    pltpu.sync_copy(thr_hbm, thr_s)
    pltpu.sync_copy(m_hbm, m_s)

    def do_row(row):
        pltpu.sync_copy(ref_hbm.at[row], refrow_v)
        rowvec = jnp.full((16,), row, jnp.int32)
        thr = plsc.load_gather(thr_s, [rowvec])
        mm = plsc.load_gather(m_s, [rowvec])

        def chunk(c, off):
            v = refrow_v[pl.ds(c * 16, 16)]
            b = jax.lax.bitcast_convert_type(v, jnp.int32)
            key = jnp.where(b >= 0, b,
                            jnp.bitwise_xor(jnp.bitwise_not(b), _MININT))
            cols = jax.lax.iota(jnp.int32, 16) + c * 16
            sel = (key > thr) | ((key == thr) & (cols < mm))
            plsc.store_compressed(idx_v.at[pl.ds(off, 16)], cols, mask=sel)
            return off + jnp.sum(sel.astype(jnp.int32))

        jax.lax.fori_loop(0, NK // 16, chunk, jnp.int32(0))
        pltpu.sync_copy(idx_v, idx_hbm.at[row])

        def gth(h, buf, sem):
            return pltpu.async_copy(
                kw1_hbm.at[idx_v.at[pl.ds(h * GCH, GCH)]], buf, sem)

        def wout(h, buf, sem):
            return pltpu.async_copy(
                buf, gath_hbm.at[pl.ds(row * K + h * GCH, GCH)], sem)

        g0 = gth(0, gbuf_a, sga)
        g1 = gth(1, gbuf_b, sgb)
        g0.wait()
        w0 = wout(0, gbuf_a, swa)
        g1.wait()
        w1 = wout(1, gbuf_b, swb)
        w0.wait()
        g2 = gth(2, gbuf_a, sga)
        w1.wait()
        g3 = gth(3, gbuf_b, sgb)
        g2.wait()
        w2 = wout(2, gbuf_a, swa)
        g3.wait()
        w3 = wout(3, gbuf_b, swb)
        w2.wait()
        w3.wait()

    for rb in range(4):
        row = wid + rb * NW

        @pl.when(row < NQP)
        def _():
            do_row(row)


def _sc_scatter_kernel(seg_hbm, ref_hbm, idx_hbm, ds_hbm, dr_hbm,
                       segout_hbm, refout_hbm,
                       segrow_v, refrow_v, idxrow_v, dsrow_v, drrow_v, sem):
    cid = jax.lax.axis_index("c")
    sid = jax.lax.axis_index("s")
    wid = sid * 2 + cid

    def do_row(row):
        pltpu.sync_copy(seg_hbm.at[row], segrow_v)
        pltpu.sync_copy(ref_hbm.at[row], refrow_v)
        pltpu.sync_copy(idx_hbm.at[row], idxrow_v)
        pltpu.sync_copy(ds_hbm.at[row], dsrow_v)
        pltpu.sync_copy(dr_hbm.at[row], drrow_v)

        def chunk(c, carry):
            iv = idxrow_v[pl.ds(c * 16, 16)]
            dsv = dsrow_v[pl.ds(c * 16, 16)]
            drv = drrow_v[pl.ds(c * 16, 16)] - 1.0e4
            plsc.addupdate_scatter(segrow_v, [iv], dsv)
            plsc.addupdate_scatter(refrow_v, [iv], drv)
            return carry

        jax.lax.fori_loop(0, K // 16, chunk, jnp.int32(0))
        pltpu.sync_copy(segrow_v, segout_hbm.at[row])
        pltpu.sync_copy(refrow_v, refout_hbm.at[row])

    for rb in range(4):
        row = wid + rb * NW

        @pl.when(row < NQP)
        def _():
            do_row(row)


def kernel(qry_feats, key_feats, Wq, Wk, Wr, W1, b1, W2s, W2r):
    f32 = jnp.float32
    i32 = jnp.int32
    qry_p = jnp.pad(qry_feats, ((0, NQP - NQ), (0, 0)))
    b1r = b1.reshape(1, D)
    w2c = jnp.pad(jnp.concatenate([W2s, W2r], axis=1).T, ((0, 6), (0, 0)))
    w2lo = w2c[:, :128]
    w2hi = w2c[:, 128:]

    q, r, qw1 = pl.pallas_call(
        _qproj_kernel,
        out_shape=[jax.ShapeDtypeStruct((NQP, D), f32)] * 3,
    )(qry_p, Wq, Wr, W1, b1r)

    seg, ref, kw1 = pl.pallas_call(
        _keyproj_kernel,
        grid=(NBLK,),
        in_specs=[
            pl.BlockSpec((BJ, D), lambda j: (j, 0)),
            pl.BlockSpec((D, D), lambda j: (0, 0)),
            pl.BlockSpec((D, D), lambda j: (0, 0)),
            pl.BlockSpec((NQP, D), lambda j: (0, 0)),
            pl.BlockSpec((NQP, D), lambda j: (0, 0)),
        ],
        out_specs=[
            pl.BlockSpec((NQP, BJ), lambda j: (0, j)),
            pl.BlockSpec((NQP, BJ), lambda j: (0, j)),
            pl.BlockSpec((BJ, D // 2), lambda j: (j, 0)),
        ],
        out_shape=[
            jax.ShapeDtypeStruct((NQP, NK), f32),
            jax.ShapeDtypeStruct((NQP, NK), f32),
            jax.ShapeDtypeStruct((NK, D // 2), jnp.int32),
        ],
    )(key_feats, Wk, W1, q, r)

    vmesh = plsc.VectorSubcoreMesh(core_axis_name="c", subcore_axis_name="s")
    sc_params = pltpu.CompilerParams(needs_layout_passes=False)

    sc_gather = pl.kernel(
        _sc_gather_kernel,
        compiler_params=sc_params,
        out_type=[
            jax.ShapeDtypeStruct((NQP, K), i32),
            jax.ShapeDtypeStruct((NQP * K, D // 2), i32),
        ],
        mesh=vmesh,
        scratch_types=[
            pltpu.VMEM((NK,), f32),
            pltpu.VMEM((K,), i32),
            pltpu.VMEM((GCH, D // 2), i32),
            pltpu.VMEM((GCH, D // 2), i32),
            pltpu.VMEM((NQP,), i32),
            pltpu.VMEM((NQP,), i32),
            pltpu.SemaphoreType.DMA,
            pltpu.SemaphoreType.DMA,
            pltpu.SemaphoreType.DMA,
            pltpu.SemaphoreType.DMA,
        ],
    )

    sc_scatter = pl.kernel(
        _sc_scatter_kernel,
        compiler_params=sc_params,
        out_type=[
            jax.ShapeDtypeStruct((NQP, NK), f32),
            jax.ShapeDtypeStruct((NQP, NK), f32),
        ],
        mesh=vmesh,
        scratch_types=[
            pltpu.VMEM((NK,), f32),
            pltpu.VMEM((NK,), f32),
            pltpu.VMEM((K,), i32),
            pltpu.VMEM((K,), f32),
            pltpu.VMEM((K,), f32),
            pltpu.SemaphoreType.DMA,
        ],
    )

    for _ in range(ITERS):
        thr, m = pl.pallas_call(
            _thresh_kernel,
            out_shape=[jax.ShapeDtypeStruct((NQP, 1), i32)] * 2,
        )(ref)

        idx, gath = sc_gather(ref, thr.reshape(NQP), m.reshape(NQP), kw1)

        ds3, dr3 = pl.pallas_call(
            _mlp_kernel,
            grid=(NQP,),
            in_specs=[
                pl.BlockSpec((K, D // 2), lambda i: (i, 0)),
                pl.BlockSpec((NQP, D), lambda i: (0, 0)),
                pl.BlockSpec((8, D // 2), lambda i: (0, 0)),
                pl.BlockSpec((8, D // 2), lambda i: (0, 0)),
            ],
            out_specs=[
                pl.BlockSpec((1, 1, K), lambda i: (i, 0, 0)),
                pl.BlockSpec((1, 1, K), lambda i: (i, 0, 0)),
            ],
            out_shape=[
                jax.ShapeDtypeStruct((NQP, 1, K), f32),
                jax.ShapeDtypeStruct((NQP, 1, K), f32),
            ],
        )(gath, qw1, w2lo, w2hi)

        seg, ref = sc_scatter(seg, ref, idx,
                              ds3.reshape(NQP, K), dr3.reshape(NQP, K))

    return seg[:NQ]


# speedup vs baseline: 7.8584x; 1.3508x over previous
import jax
import jax.numpy as jnp
import numpy as np
from jax.experimental import pallas as pl
from jax.experimental.pallas import tpu as pltpu
from jax.experimental.pallas import tpu_sc as plsc

NQ = 100
NK = 16384
D = 256
K = 1024
ITERS = 2

NQP = 104
BJ = 1024
NBLK = NK // BJ
NW = 32
GCH = 256

_MININT = np.int32(-2147483648)


def _sortkey(x):
    b = jax.lax.bitcast_convert_type(x, jnp.int32)
    return jnp.where(b >= 0, b, jnp.bitwise_xor(jnp.bitwise_not(b), _MININT))


def _qproj_kernel(qry_ref, wq_ref, wr_ref, w1_ref, b1_ref, q_ref, r_ref, qw1_ref):
    qf = qry_ref[...]
    q = jnp.dot(qf, wq_ref[...], preferred_element_type=jnp.float32)
    q_ref[...] = q
    r_ref[...] = jnp.dot(qf, wr_ref[...], preferred_element_type=jnp.float32)
    qw1_ref[...] = (
        jnp.dot(q, w1_ref[...], preferred_element_type=jnp.float32) + b1_ref[...]
    )


def _keyproj_kernel(key_ref, wk_ref, w1_ref, q_ref, r_ref,
                    seg_ref, ref_ref, kw1p_ref):
    kb = jnp.dot(key_ref[...], wk_ref[...], preferred_element_type=jnp.float32)
    dn = (((1,), (1,)), ((), ()))
    seg_ref[...] = jax.lax.dot_general(q_ref[...], kb, dn,
                                       preferred_element_type=jnp.float32)
    ref_ref[...] = jax.lax.dot_general(r_ref[...], kb, dn,
                                       preferred_element_type=jnp.float32)
    kw1 = jnp.dot(kb, w1_ref[...], preferred_element_type=jnp.float32)
    ilo = jax.lax.bitcast_convert_type(kw1[:, :128], jnp.int32)
    ihi = jax.lax.bitcast_convert_type(kw1[:, 128:], jnp.int32)
    half = jnp.int32(0x8000)
    kw1p_ref[...] = (((ilo + half) >> 16) & jnp.int32(0xFFFF)) | (
        (ihi + half) & jnp.int32(-65536))


def _count_ge(keys, cand_s):
    return jnp.sum((keys >= cand_s).astype(jnp.int32), axis=1, keepdims=True)


def _search(keys, col, kk):
    def bit_step(i, tu):
        b = 31 - i
        cand = tu | (jnp.int32(1) << b)
        cnt = _count_ge(keys, cand ^ _MININT)
        return jnp.where(cnt >= kk, cand, tu)

    tu = jax.lax.fori_loop(0, 32, bit_step, jnp.zeros(keys.shape[:1] + (1,),
                                                      jnp.int32))
    ts = tu ^ _MININT
    cnt_gt = jnp.sum((keys > ts).astype(jnp.int32), axis=1, keepdims=True)
    need = kk - cnt_gt
    eqm = keys == ts

    def m_step(i, lo_hi):
        lo, hi = lo_hi
        mid = (lo + hi) // 2
        c = jnp.sum((eqm & (col < mid)).astype(jnp.int32), axis=1, keepdims=True)
        take = c >= need
        return jnp.where(take, lo, mid + 1), jnp.where(take, mid, hi)

    z = jnp.zeros(keys.shape[:1] + (1,), jnp.int32)
    _, hi = jax.lax.fori_loop(0, 15, m_step, (z, jnp.full_like(z, NK)))
    return ts, hi


def _thresh_kernel(ref_ref, thr1_ref, m1_ref, thr2_ref, m2_ref):
    keys = _sortkey(ref_ref[...])
    col = jax.lax.broadcasted_iota(jnp.int32, (NQP, NK), 1)
    thr1_ref[...], m1_ref[...] = _search(keys, col, K)
    thr2_ref[...], m2_ref[...] = _search(keys, col, 2 * K)


def _mlp_kernel(gath_ref, qw1_ref, w2lo_ref, w2hi_ref, ds_ref):
    i = pl.program_id(0)
    qrow = qw1_ref[pl.ds(i, 1), :]
    packed = gath_ref[...]
    x_lo = jax.lax.bitcast_convert_type(packed << 16, jnp.float32)
    x_hi = jax.lax.bitcast_convert_type(packed & jnp.int32(-65536),
                                        jnp.float32)
    h_lo = jnp.tanh(x_lo + qrow[:, :128]).astype(jnp.bfloat16)
    h_hi = jnp.tanh(x_hi + qrow[:, 128:]).astype(jnp.bfloat16)
    dn = (((1,), (1,)), ((), ()))
    y = (jax.lax.dot_general(h_lo, w2lo_ref[...].astype(jnp.bfloat16), dn,
                             preferred_element_type=jnp.float32)
         + jax.lax.dot_general(h_hi, w2hi_ref[...].astype(jnp.bfloat16), dn,
                               preferred_element_type=jnp.float32))
    ds_ref[...] = jnp.transpose(y[:, 0:1]).reshape(1, 1, K)


def _make_sc_gather(second):
    def body(ref_hbm, thr1_hbm, m1_hbm, thr2_hbm, m2_hbm, kw1_hbm,
             idx_hbm, gath_hbm,
             refrow_v, idx_v, gbuf_a, gbuf_b, thr1_s, m1_s, thr2_s, m2_s,
             sga, sgb, swa, swb):
        cid = jax.lax.axis_index("c")
        sid = jax.lax.axis_index("s")
        wid = sid * 2 + cid
        pltpu.sync_copy(thr1_hbm, thr1_s)
        pltpu.sync_copy(m1_hbm, m1_s)
        if second:
            pltpu.sync_copy(thr2_hbm, thr2_s)
            pltpu.sync_copy(m2_hbm, m2_s)

        def do_row(row):
            pltpu.sync_copy(ref_hbm.at[row], refrow_v)
            rowvec = jnp.full((16,), row, jnp.int32)
            thr1 = plsc.load_gather(thr1_s, [rowvec])
            mm1 = plsc.load_gather(m1_s, [rowvec])
            if second:
                thr2 = plsc.load_gather(thr2_s, [rowvec])
                mm2 = plsc.load_gather(m2_s, [rowvec])

            def chunk(c, off):
                v = refrow_v[pl.ds(c * 16, 16)]
                b = jax.lax.bitcast_convert_type(v, jnp.int32)
                key = jnp.where(b >= 0, b,
                                jnp.bitwise_xor(jnp.bitwise_not(b), _MININT))
                cols = jax.lax.iota(jnp.int32, 16) + c * 16
                s1 = (key > thr1) | ((key == thr1) & (cols < mm1))
                if second:
                    s2 = (key > thr2) | ((key == thr2) & (cols < mm2))
                    sel = s2 & jnp.logical_not(s1)
                else:
                    sel = s1
                plsc.store_compressed(idx_v.at[pl.ds(off, 16)], cols, mask=sel)
                return off + jnp.sum(sel.astype(jnp.int32))

            jax.lax.fori_loop(0, NK // 16, chunk, jnp.int32(0))
            pltpu.sync_copy(idx_v, idx_hbm.at[row])

            def gth(h, buf, sem):
                return pltpu.async_copy(
                    kw1_hbm.at[idx_v.at[pl.ds(h * GCH, GCH)]], buf, sem)

            def wout(h, buf, sem):
                return pltpu.async_copy(
                    buf, gath_hbm.at[pl.ds(row * K + h * GCH, GCH)], sem)

            g0 = gth(0, gbuf_a, sga)
            g1 = gth(1, gbuf_b, sgb)
            g0.wait()
            w0 = wout(0, gbuf_a, swa)
            g1.wait()
            w1 = wout(1, gbuf_b, swb)
            w0.wait()
            g2 = gth(2, gbuf_a, sga)
            w1.wait()
            g3 = gth(3, gbuf_b, sgb)
            g2.wait()
            w2 = wout(2, gbuf_a, swa)
            g3.wait()
            w3 = wout(3, gbuf_b, swb)
            w2.wait()
            w3.wait()

        for rb in range(4):
            row = wid + rb * NW

            @pl.when(row < NQ)
            def _():
                do_row(row)

    return body


def _sc_scatter_kernel(seg_hbm, idx1_hbm, ds1_hbm, idx2_hbm, ds2_hbm,
                       segout_hbm,
                       segrow_v, idxrow_v, dsrow_v, idxrow2_v, dsrow2_v, sem):
    cid = jax.lax.axis_index("c")
    sid = jax.lax.axis_index("s")
    wid = sid * 2 + cid

    def do_row(row):
        pltpu.sync_copy(seg_hbm.at[row], segrow_v)
        pltpu.sync_copy(idx1_hbm.at[row], idxrow_v)
        pltpu.sync_copy(ds1_hbm.at[row], dsrow_v)
        pltpu.sync_copy(idx2_hbm.at[row], idxrow2_v)
        pltpu.sync_copy(ds2_hbm.at[row], dsrow2_v)

        def chunk(c, carry):
            iv = idxrow_v[pl.ds(c * 16, 16)]
            plsc.addupdate_scatter(segrow_v, [iv], dsrow_v[pl.ds(c * 16, 16)])
            iv2 = idxrow2_v[pl.ds(c * 16, 16)]
            plsc.addupdate_scatter(segrow_v, [iv2], dsrow2_v[pl.ds(c * 16, 16)])
            return carry

        jax.lax.fori_loop(0, K // 16, chunk, jnp.int32(0))
        pltpu.sync_copy(segrow_v, segout_hbm.at[row])

    for rb in range(4):
        row = wid + rb * NW

        @pl.when(row < NQ)
        def _():
            do_row(row)


def kernel(qry_feats, key_feats, Wq, Wk, Wr, W1, b1, W2s, W2r):
    f32 = jnp.float32
    i32 = jnp.int32
    qry_p = jnp.pad(qry_feats, ((0, NQP - NQ), (0, 0)))
    b1r = b1.reshape(1, D)
    w2c = jnp.pad(jnp.concatenate([W2s, W2r], axis=1).T, ((0, 6), (0, 0)))
    w2lo = w2c[:, :128]
    w2hi = w2c[:, 128:]

    q, r, qw1 = pl.pallas_call(
        _qproj_kernel,
        out_shape=[jax.ShapeDtypeStruct((NQP, D), f32)] * 3,
    )(qry_p, Wq, Wr, W1, b1r)

    seg, ref, kw1p = pl.pallas_call(
        _keyproj_kernel,
        grid=(NBLK,),
        in_specs=[
            pl.BlockSpec((BJ, D), lambda j: (j, 0)),
            pl.BlockSpec((D, D), lambda j: (0, 0)),
            pl.BlockSpec((D, D), lambda j: (0, 0)),
            pl.BlockSpec((NQP, D), lambda j: (0, 0)),
            pl.BlockSpec((NQP, D), lambda j: (0, 0)),
        ],
        out_specs=[
            pl.BlockSpec((NQP, BJ), lambda j: (0, j)),
            pl.BlockSpec((NQP, BJ), lambda j: (0, j)),
            pl.BlockSpec((BJ, D // 2), lambda j: (j, 0)),
        ],
        out_shape=[
            jax.ShapeDtypeStruct((NQP, NK), f32),
            jax.ShapeDtypeStruct((NQP, NK), f32),
            jax.ShapeDtypeStruct((NK, D // 2), jnp.int32),
        ],
    )(key_feats, Wk, W1, q, r)

    thr1, m1, thr2, m2 = pl.pallas_call(
        _thresh_kernel,
        out_shape=[jax.ShapeDtypeStruct((NQP, 1), i32)] * 4,
    )(ref)
    targs = (thr1.reshape(NQP), m1.reshape(NQP),
             thr2.reshape(NQP), m2.reshape(NQP))

    vmesh = plsc.VectorSubcoreMesh(core_axis_name="c", subcore_axis_name="s")
    sc_params = pltpu.CompilerParams(needs_layout_passes=False)

    gather_out = [
        jax.ShapeDtypeStruct((NQ, K), i32),
        jax.ShapeDtypeStruct((NQ * K, D // 2), i32),
    ]
    gather_scratch = [
        pltpu.VMEM((NK,), f32),
        pltpu.VMEM((K,), i32),
        pltpu.VMEM((GCH, D // 2), i32),
        pltpu.VMEM((GCH, D // 2), i32),
        pltpu.VMEM((NQP,), i32),
        pltpu.VMEM((NQP,), i32),
        pltpu.VMEM((NQP,), i32),
        pltpu.VMEM((NQP,), i32),
        pltpu.SemaphoreType.DMA,
        pltpu.SemaphoreType.DMA,
        pltpu.SemaphoreType.DMA,
        pltpu.SemaphoreType.DMA,
    ]

    idx1, gath1 = pl.kernel(
        _make_sc_gather(False), compiler_params=sc_params,
        out_type=gather_out, mesh=vmesh, scratch_types=gather_scratch,
    )(ref, *targs, kw1p)
    idx2, gath2 = pl.kernel(
        _make_sc_gather(True), compiler_params=sc_params,
        out_type=gather_out, mesh=vmesh, scratch_types=gather_scratch,
    )(ref, *targs, kw1p)

    def run_mlp(gath):
        (ds3,) = pl.pallas_call(
            _mlp_kernel,
            grid=(NQ,),
            in_specs=[
                pl.BlockSpec((K, D // 2), lambda i: (i, 0)),
                pl.BlockSpec((NQP, D), lambda i: (0, 0)),
                pl.BlockSpec((8, D // 2), lambda i: (0, 0)),
                pl.BlockSpec((8, D // 2), lambda i: (0, 0)),
            ],
            out_specs=[
                pl.BlockSpec((1, 1, K), lambda i: (i, 0, 0)),
            ],
            out_shape=[
                jax.ShapeDtypeStruct((NQ, 1, K), f32),
            ],
        )(gath, qw1, w2lo, w2hi)
        return ds3.reshape(NQ, K)

    ds1 = run_mlp(gath1)
    ds2 = run_mlp(gath2)

    segout = pl.kernel(
        _sc_scatter_kernel, compiler_params=sc_params,
        out_type=jax.ShapeDtypeStruct((NQP, NK), f32),
        mesh=vmesh,
        scratch_types=[
            pltpu.VMEM((NK,), f32),
            pltpu.VMEM((K,), i32),
            pltpu.VMEM((K,), f32),
            pltpu.VMEM((K,), i32),
            pltpu.VMEM((K,), f32),
            pltpu.SemaphoreType.DMA,
        ],
    )(seg, idx1, ds1, idx2, ds2)

    return segout[:NQ]
